# Initial kernel scaffold; baseline (speedup 1.0000x reference)
import jax
import jax.numpy as jnp
from jax.experimental import pallas as pl


def kernel(seg_logits, seg_labels, align_logits, align_labels, rec_pred, rec_target):
    raise NotImplementedError("write your pallas kernel here")


# speedup vs baseline: 67.2491x; 67.2491x over previous
import functools

import jax
import jax.numpy as jnp
from jax import lax
from jax.experimental import pallas as pl
from jax.experimental.pallas import tpu as pltpu
from jax.experimental.pallas import tpu_sc as plsc

_K = 1024
_BINS = 2 * _K
_NTILES = 32
_CHUNK = 65536
_NPIX = 8 * 512 * 512


def _tc1_body(seg_ref, lab_ref, rp_ref, rt_ref, cid_ref, part_ref):
    b = pl.program_id(0)
    q = pl.program_id(1)
    d = seg_ref[0, 1] - seg_ref[0, 0]
    lab = lab_ref[0]
    z = jnp.where(lab == 1, -d, d)
    sp = jnp.maximum(z, 0.0) + jnp.log1p(jnp.exp(-jnp.abs(z)))
    nll = sp
    smooth = 2.0 * sp - z
    e = jnp.exp(z - sp)
    bi = jnp.minimum((e * _K).astype(jnp.int32), _K - 1)
    bi = jnp.maximum(bi, 0)
    cid_ref[0] = bi + _K * lab
    s_nll = jnp.sum(nll)
    s_sm = jnp.sum(smooth)
    s_rec = jnp.sum(jnp.abs(rp_ref[0] - rt_ref[0]))
    first = jnp.logical_and(b == 0, q == 0)

    @pl.when(first)
    def _():
        part_ref[0, 0] = s_nll
        part_ref[0, 1] = s_sm
        part_ref[0, 2] = s_rec
        part_ref[0, 3] = 0.0

    @pl.when(jnp.logical_not(first))
    def _():
        part_ref[0, 0] += s_nll
        part_ref[0, 1] += s_sm
        part_ref[0, 2] += s_rec


def _sc_hist_body(cid_hbm, out_hbm, inbuf, hist, outbuf):
    wid = lax.axis_index("s") * 2 + lax.axis_index("c")
    pltpu.sync_copy(cid_hbm.at[wid], inbuf)
    zeros = jnp.zeros((16,), jnp.float32)
    ones = jnp.ones((16,), jnp.float32)
    lane_base = lax.iota(jnp.int32, 16) * _BINS

    def zbody(i, c):
        hist[pl.ds(i * 16, 16)] = zeros
        return c

    lax.fori_loop(0, (16 * _BINS) // 16, zbody, 0)

    def sbody(i, c):
        v = inbuf[pl.ds(i * 16, 16)]
        plsc.addupdate_scatter(hist, [v + lane_base], ones)
        return c

    lax.fori_loop(0, _CHUNK // 16, sbody, 0)

    def rbody(cch, c):
        acc = hist[pl.ds(cch * 16, 16)]
        for l in range(1, 16):
            acc = acc + hist[pl.ds(l * _BINS + cch * 16, 16)]
        outbuf[pl.ds(cch * 16, 16)] = acc
        return c

    lax.fori_loop(0, _BINS // 16, rbody, 0)
    pltpu.sync_copy(outbuf, out_hbm.at[wid])


@functools.cache
def _sc_hist():
    return pl.kernel(
        _sc_hist_body,
        out_type=jax.ShapeDtypeStruct((_NTILES, _BINS), jnp.float32),
        mesh=plsc.VectorSubcoreMesh(core_axis_name="c", subcore_axis_name="s"),
        scratch_types=[
            pltpu.VMEM((_CHUNK,), jnp.int32),
            pltpu.VMEM((16 * _BINS,), jnp.float32),
            pltpu.VMEM((_BINS,), jnp.float32),
        ],
        compiler_params=pltpu.CompilerParams(needs_layout_passes=False),
    )


def _tc2_body(hist_ref, part_ref, al_ref, lab_ref, out_ref):
    h = hist_ref[...]
    dot = functools.partial(
        jax.lax.dot_general,
        precision=jax.lax.Precision.HIGHEST,
        preferred_element_type=jnp.float32,
    )
    ti = lax.broadcasted_iota(jnp.int32, (8, _NTILES), 1)
    ri = lax.broadcasted_iota(jnp.int32, (8, _NTILES), 0)
    m = (ti // 4 == ri).astype(jnp.float32)
    him = dot(m, h, (((1,), (0,)), ((), ())))
    h1 = him[:, _K:]
    n = him[:, :_K] + h1
    jj = lax.broadcasted_iota(jnp.int32, (_K, _K), 0)
    kk = lax.broadcasted_iota(jnp.int32, (_K, _K), 1)
    u = (jj >= kk).astype(jnp.float32)
    i_cum = dot(n, u, (((1,), (0,)), ((), ())))
    f1 = dot(h1, u, (((1,), (0,)), ((), ())))
    p1 = f1[:, 0:1]
    ptot = i_cum[:, 0:1]

    def loss_for(f, p):
        denom = jnp.maximum(p + i_cum - f, 1.0)
        jac = 1.0 - (p - f) / denom
        jac = jnp.where(i_cum > 0, jac, 0.0)
        return (jnp.sum(jac, axis=1, keepdims=True) - 0.5 * jac[:, 0:1]) / _K

    l1 = loss_for(f1, p1)
    l0 = loss_for(i_cum - f1, ptot - p1)
    pr1 = (p1 > 0).astype(jnp.float32)
    pr0 = (ptot - p1 > 0).astype(jnp.float32)
    per_img = (pr0 * l0 + pr1 * l1) / jnp.maximum(pr0 + pr1, 1.0)
    lovasz = jnp.sum(per_img) / 8.0
    a = al_ref[...]
    mx = jnp.max(a, axis=1, keepdims=True)
    lse = jnp.log(jnp.sum(jnp.exp(a - mx), axis=1, keepdims=True)) + mx
    logp = a - lse
    colj = lax.broadcasted_iota(jnp.int32, (8, 8), 1)
    pick = jnp.sum(jnp.where(colj == lab_ref[...], logp, 0.0), axis=1)
    align_ce = -jnp.sum(pick) / 8.0
    seg_ce = 0.9 * (part_ref[0, 0] / _NPIX) + 0.05 * (part_ref[0, 1] / _NPIX)
    rec = part_ref[0, 2] / (3 * _NPIX)
    out_ref[0, 0] = 3.0 * (seg_ce + lovasz) + align_ce + rec


def kernel(seg_logits, seg_labels, align_logits, align_labels, rec_pred, rec_target):
    cid, partials = pl.pallas_call(
        _tc1_body,
        grid=(8, 4),
        in_specs=[
            pl.BlockSpec((1, 2, 128, 512), lambda b, q: (b, 0, q, 0)),
            pl.BlockSpec((1, 128, 512), lambda b, q: (b, q, 0)),
            pl.BlockSpec((1, 3, 128, 512), lambda b, q: (b, 0, q, 0)),
            pl.BlockSpec((1, 3, 128, 512), lambda b, q: (b, 0, q, 0)),
        ],
        out_specs=[
            pl.BlockSpec((1, 128, 512), lambda b, q: (b, q, 0)),
            pl.BlockSpec(memory_space=pltpu.SMEM),
        ],
        out_shape=[
            jax.ShapeDtypeStruct((8, 512, 512), jnp.int32),
            jax.ShapeDtypeStruct((1, 4), jnp.float32),
        ],
    )(seg_logits, seg_labels.astype(jnp.int32), rec_pred, rec_target)

    hist = _sc_hist()(cid.reshape(_NTILES, _CHUNK))

    total = pl.pallas_call(
        _tc2_body,
        in_specs=[
            pl.BlockSpec(memory_space=pltpu.VMEM),
            pl.BlockSpec(memory_space=pltpu.SMEM),
            pl.BlockSpec(memory_space=pltpu.VMEM),
            pl.BlockSpec(memory_space=pltpu.VMEM),
        ],
        out_specs=pl.BlockSpec(memory_space=pltpu.SMEM),
        out_shape=jax.ShapeDtypeStruct((1, 1), jnp.float32),
    )(hist, partials, align_logits, align_labels.astype(jnp.int32).reshape(8, 1))
    return total.reshape(())


# speedup vs baseline: 72.7515x; 1.0818x over previous
import functools

import jax
import jax.numpy as jnp
from jax import lax
from jax.experimental import pallas as pl
from jax.experimental.pallas import tpu as pltpu
from jax.experimental.pallas import tpu_sc as plsc

_K = 1024
_BINS = 2 * _K
_NTILES = 32
_CHUNK = 65536
_NPIX = 8 * 512 * 512


def _tc1_body(seg_ref, lab_ref, rp_ref, rt_ref, cid_ref, part_ref):
    b = pl.program_id(0)
    q = pl.program_id(1)
    d = seg_ref[0, 1] - seg_ref[0, 0]
    lab = lab_ref[0]
    z = jnp.where(lab == 1, -d, d)
    sp = jnp.maximum(z, 0.0) + jnp.log1p(jnp.exp(-jnp.abs(z)))
    nll = sp
    smooth = 2.0 * sp - z
    e = jnp.exp(z - sp)
    bi = jnp.minimum((e * _K).astype(jnp.int32), _K - 1)
    bi = jnp.maximum(bi, 0)
    cid_ref[0] = bi + _K * lab
    s_nll = jnp.sum(nll)
    s_sm = jnp.sum(smooth)
    s_rec = jnp.sum(jnp.abs(rp_ref[0] - rt_ref[0]))
    first = jnp.logical_and(b == 0, q == 0)

    @pl.when(first)
    def _():
        part_ref[0, 0] = s_nll
        part_ref[0, 1] = s_sm
        part_ref[0, 2] = s_rec
        part_ref[0, 3] = 0.0

    @pl.when(jnp.logical_not(first))
    def _():
        part_ref[0, 0] += s_nll
        part_ref[0, 1] += s_sm
        part_ref[0, 2] += s_rec


def _sc_hist_body(cid_hbm, out_hbm, inbuf, hist, outbuf):
    wid = lax.axis_index("s") * 2 + lax.axis_index("c")
    pltpu.sync_copy(cid_hbm.at[wid], inbuf)
    zeros = jnp.zeros((16,), jnp.float32)
    ones = jnp.ones((16,), jnp.float32)
    lane_base = lax.iota(jnp.int32, 16) * _BINS

    def zbody(i, c):
        for u in range(8):
            hist[pl.ds(i * 128 + u * 16, 16)] = zeros
        return c

    lax.fori_loop(0, (16 * _BINS) // 128, zbody, 0)

    def sbody(i, c):
        for u in range(8):
            v = inbuf[pl.ds(i * 128 + u * 16, 16)]
            plsc.addupdate_scatter(hist, [v + lane_base], ones)
        return c

    lax.fori_loop(0, _CHUNK // 128, sbody, 0)

    def rbody(cch, c):
        acc = hist[pl.ds(cch * 16, 16)]
        for l in range(1, 16):
            acc = acc + hist[pl.ds(l * _BINS + cch * 16, 16)]
        outbuf[pl.ds(cch * 16, 16)] = acc
        return c

    lax.fori_loop(0, _BINS // 16, rbody, 0)
    pltpu.sync_copy(outbuf, out_hbm.at[wid])


@functools.cache
def _sc_hist():
    return pl.kernel(
        _sc_hist_body,
        out_type=jax.ShapeDtypeStruct((_NTILES, _BINS), jnp.float32),
        mesh=plsc.VectorSubcoreMesh(core_axis_name="c", subcore_axis_name="s"),
        scratch_types=[
            pltpu.VMEM((_CHUNK,), jnp.int32),
            pltpu.VMEM((16 * _BINS,), jnp.float32),
            pltpu.VMEM((_BINS,), jnp.float32),
        ],
        compiler_params=pltpu.CompilerParams(needs_layout_passes=False),
    )


def _tc2_body(hist_ref, part_ref, al_ref, lab_ref, out_ref):
    h = hist_ref[...]
    dot = functools.partial(
        jax.lax.dot_general,
        precision=jax.lax.Precision.HIGHEST,
        preferred_element_type=jnp.float32,
    )
    ti = lax.broadcasted_iota(jnp.int32, (8, _NTILES), 1)
    ri = lax.broadcasted_iota(jnp.int32, (8, _NTILES), 0)
    m = (ti // 4 == ri).astype(jnp.float32)
    him = dot(m, h, (((1,), (0,)), ((), ())))
    h1 = him[:, _K:]
    n = him[:, :_K] + h1
    jj = lax.broadcasted_iota(jnp.int32, (_K, _K), 0)
    kk = lax.broadcasted_iota(jnp.int32, (_K, _K), 1)
    u = (jj >= kk).astype(jnp.float32)
    i_cum = dot(n, u, (((1,), (0,)), ((), ())))
    f1 = dot(h1, u, (((1,), (0,)), ((), ())))
    p1 = f1[:, 0:1]
    ptot = i_cum[:, 0:1]

    def loss_for(f, p):
        denom = jnp.maximum(p + i_cum - f, 1.0)
        jac = 1.0 - (p - f) / denom
        jac = jnp.where(i_cum > 0, jac, 0.0)
        return (jnp.sum(jac, axis=1, keepdims=True) - 0.5 * jac[:, 0:1]) / _K

    l1 = loss_for(f1, p1)
    l0 = loss_for(i_cum - f1, ptot - p1)
    pr1 = (p1 > 0).astype(jnp.float32)
    pr0 = (ptot - p1 > 0).astype(jnp.float32)
    per_img = (pr0 * l0 + pr1 * l1) / jnp.maximum(pr0 + pr1, 1.0)
    lovasz = jnp.sum(per_img) / 8.0
    a = al_ref[...]
    mx = jnp.max(a, axis=1, keepdims=True)
    lse = jnp.log(jnp.sum(jnp.exp(a - mx), axis=1, keepdims=True)) + mx
    logp = a - lse
    colj = lax.broadcasted_iota(jnp.int32, (8, 8), 1)
    pick = jnp.sum(jnp.where(colj == lab_ref[...], logp, 0.0), axis=1)
    align_ce = -jnp.sum(pick) / 8.0
    seg_ce = 0.9 * (part_ref[0, 0] / _NPIX) + 0.05 * (part_ref[0, 1] / _NPIX)
    rec = part_ref[0, 2] / (3 * _NPIX)
    out_ref[0, 0] = 3.0 * (seg_ce + lovasz) + align_ce + rec


def kernel(seg_logits, seg_labels, align_logits, align_labels, rec_pred, rec_target):
    cid, partials = pl.pallas_call(
        _tc1_body,
        grid=(8, 4),
        in_specs=[
            pl.BlockSpec((1, 2, 128, 512), lambda b, q: (b, 0, q, 0)),
            pl.BlockSpec((1, 128, 512), lambda b, q: (b, q, 0)),
            pl.BlockSpec((1, 3, 128, 512), lambda b, q: (b, 0, q, 0)),
            pl.BlockSpec((1, 3, 128, 512), lambda b, q: (b, 0, q, 0)),
        ],
        out_specs=[
            pl.BlockSpec((1, 128, 512), lambda b, q: (b, q, 0)),
            pl.BlockSpec(memory_space=pltpu.SMEM),
        ],
        out_shape=[
            jax.ShapeDtypeStruct((8, 512, 512), jnp.int32),
            jax.ShapeDtypeStruct((1, 4), jnp.float32),
        ],
    )(seg_logits, seg_labels.astype(jnp.int32), rec_pred, rec_target)

    hist = _sc_hist()(cid.reshape(_NTILES, _CHUNK))

    total = pl.pallas_call(
        _tc2_body,
        in_specs=[
            pl.BlockSpec(memory_space=pltpu.VMEM),
            pl.BlockSpec(memory_space=pltpu.SMEM),
            pl.BlockSpec(memory_space=pltpu.VMEM),
            pl.BlockSpec(memory_space=pltpu.VMEM),
        ],
        out_specs=pl.BlockSpec(memory_space=pltpu.SMEM),
        out_shape=jax.ShapeDtypeStruct((1, 1), jnp.float32),
    )(hist, partials, align_logits, align_labels.astype(jnp.int32).reshape(8, 1))
    return total.reshape(())


# speedup vs baseline: 91.1128x; 1.2524x over previous
import functools

import jax
import jax.numpy as jnp
from jax import lax
from jax.experimental import pallas as pl
from jax.experimental.pallas import tpu as pltpu
from jax.experimental.pallas import tpu_sc as plsc

_K = 1024
_BINS = 2 * _K
_NTILES = 32
_CHUNK = 65536
_NPIX = 8 * 512 * 512


def _tc1_body(seg_ref, lab_ref, rp_ref, rt_ref, cid_ref, part_ref):
    b = pl.program_id(0)
    q = pl.program_id(1)
    d = seg_ref[0, 1] - seg_ref[0, 0]
    lab = lab_ref[0]
    z = jnp.where(lab == 1, -d, d)
    sp = jnp.maximum(z, 0.0) + jnp.log1p(jnp.exp(-jnp.abs(z)))
    nll = sp
    smooth = 2.0 * sp - z
    e = jnp.exp(z - sp)
    bi = jnp.minimum((e * _K).astype(jnp.int32), _K - 1)
    bi = jnp.maximum(bi, 0)
    cid_ref[0] = bi + _K * lab
    s_nll = jnp.sum(nll)
    s_sm = jnp.sum(smooth)
    s_rec = jnp.sum(jnp.abs(rp_ref[0] - rt_ref[0]))
    first = jnp.logical_and(b == 0, q == 0)

    @pl.when(first)
    def _():
        part_ref[0, 0] = s_nll
        part_ref[0, 1] = s_sm
        part_ref[0, 2] = s_rec
        part_ref[0, 3] = 0.0

    @pl.when(jnp.logical_not(first))
    def _():
        part_ref[0, 0] += s_nll
        part_ref[0, 1] += s_sm
        part_ref[0, 2] += s_rec


def _sc_hist_body(cid_hbm, out_hbm, inbuf, hist, outbuf):
    wid = lax.axis_index("s") * 2 + lax.axis_index("c")
    pltpu.sync_copy(cid_hbm.at[wid], inbuf)
    zeros = jnp.zeros((16,), jnp.float32)
    ones = jnp.ones((16,), jnp.float32)
    lane_base = lax.iota(jnp.int32, 16) * _BINS

    def zbody(i, c):
        for u in range(8):
            hist[pl.ds(i * 128 + u * 16, 16)] = zeros
        return c

    lax.fori_loop(0, (16 * _BINS) // 128, zbody, 0)

    @plsc.parallel_loop(0, _CHUNK // 16, unroll=8)
    def _scatter(i):
        v = inbuf[pl.ds(i * 16, 16)]
        plsc.addupdate_scatter(hist, [v + lane_base], ones)

    def rbody(cch, c):
        acc = hist[pl.ds(cch * 16, 16)]
        for l in range(1, 16):
            acc = acc + hist[pl.ds(l * _BINS + cch * 16, 16)]
        outbuf[pl.ds(cch * 16, 16)] = acc
        return c

    lax.fori_loop(0, _BINS // 16, rbody, 0)
    pltpu.sync_copy(outbuf, out_hbm.at[wid])


@functools.cache
def _sc_hist():
    return pl.kernel(
        _sc_hist_body,
        out_type=jax.ShapeDtypeStruct((_NTILES, _BINS), jnp.float32),
        mesh=plsc.VectorSubcoreMesh(core_axis_name="c", subcore_axis_name="s"),
        scratch_types=[
            pltpu.VMEM((_CHUNK,), jnp.int32),
            pltpu.VMEM((16 * _BINS,), jnp.float32),
            pltpu.VMEM((_BINS,), jnp.float32),
        ],
        compiler_params=pltpu.CompilerParams(needs_layout_passes=False),
    )


def _tc2_body(hist_ref, part_ref, al_ref, lab_ref, out_ref):
    h = hist_ref[...]
    dot = functools.partial(
        jax.lax.dot_general,
        precision=jax.lax.Precision.HIGHEST,
        preferred_element_type=jnp.float32,
    )
    ti = lax.broadcasted_iota(jnp.int32, (8, _NTILES), 1)
    ri = lax.broadcasted_iota(jnp.int32, (8, _NTILES), 0)
    m = (ti // 4 == ri).astype(jnp.float32)
    him = dot(m, h, (((1,), (0,)), ((), ())))
    h1 = him[:, _K:]
    n = him[:, :_K] + h1
    jj = lax.broadcasted_iota(jnp.int32, (_K, _K), 0)
    kk = lax.broadcasted_iota(jnp.int32, (_K, _K), 1)
    u = (jj >= kk).astype(jnp.float32)
    i_cum = dot(n, u, (((1,), (0,)), ((), ())))
    f1 = dot(h1, u, (((1,), (0,)), ((), ())))
    p1 = f1[:, 0:1]
    ptot = i_cum[:, 0:1]

    def loss_for(f, p):
        denom = jnp.maximum(p + i_cum - f, 1.0)
        jac = 1.0 - (p - f) / denom
        jac = jnp.where(i_cum > 0, jac, 0.0)
        return (jnp.sum(jac, axis=1, keepdims=True) - 0.5 * jac[:, 0:1]) / _K

    l1 = loss_for(f1, p1)
    l0 = loss_for(i_cum - f1, ptot - p1)
    pr1 = (p1 > 0).astype(jnp.float32)
    pr0 = (ptot - p1 > 0).astype(jnp.float32)
    per_img = (pr0 * l0 + pr1 * l1) / jnp.maximum(pr0 + pr1, 1.0)
    lovasz = jnp.sum(per_img) / 8.0
    a = al_ref[...]
    mx = jnp.max(a, axis=1, keepdims=True)
    lse = jnp.log(jnp.sum(jnp.exp(a - mx), axis=1, keepdims=True)) + mx
    logp = a - lse
    colj = lax.broadcasted_iota(jnp.int32, (8, 8), 1)
    pick = jnp.sum(jnp.where(colj == lab_ref[...], logp, 0.0), axis=1)
    align_ce = -jnp.sum(pick) / 8.0
    seg_ce = 0.9 * (part_ref[0, 0] / _NPIX) + 0.05 * (part_ref[0, 1] / _NPIX)
    rec = part_ref[0, 2] / (3 * _NPIX)
    out_ref[0, 0] = 3.0 * (seg_ce + lovasz) + align_ce + rec


def kernel(seg_logits, seg_labels, align_logits, align_labels, rec_pred, rec_target):
    cid, partials = pl.pallas_call(
        _tc1_body,
        grid=(8, 4),
        in_specs=[
            pl.BlockSpec((1, 2, 128, 512), lambda b, q: (b, 0, q, 0)),
            pl.BlockSpec((1, 128, 512), lambda b, q: (b, q, 0)),
            pl.BlockSpec((1, 3, 128, 512), lambda b, q: (b, 0, q, 0)),
            pl.BlockSpec((1, 3, 128, 512), lambda b, q: (b, 0, q, 0)),
        ],
        out_specs=[
            pl.BlockSpec((1, 128, 512), lambda b, q: (b, q, 0)),
            pl.BlockSpec(memory_space=pltpu.SMEM),
        ],
        out_shape=[
            jax.ShapeDtypeStruct((8, 512, 512), jnp.int32),
            jax.ShapeDtypeStruct((1, 4), jnp.float32),
        ],
    )(seg_logits, seg_labels.astype(jnp.int32), rec_pred, rec_target)

    hist = _sc_hist()(cid.reshape(_NTILES, _CHUNK))

    total = pl.pallas_call(
        _tc2_body,
        in_specs=[
            pl.BlockSpec(memory_space=pltpu.VMEM),
            pl.BlockSpec(memory_space=pltpu.SMEM),
            pl.BlockSpec(memory_space=pltpu.VMEM),
            pl.BlockSpec(memory_space=pltpu.VMEM),
        ],
        out_specs=pl.BlockSpec(memory_space=pltpu.SMEM),
        out_shape=jax.ShapeDtypeStruct((1, 1), jnp.float32),
    )(hist, partials, align_logits, align_labels.astype(jnp.int32).reshape(8, 1))
    return total.reshape(())


# speedup vs baseline: 99.2100x; 1.0889x over previous
import functools

import jax
import jax.numpy as jnp
from jax import lax
from jax.experimental import pallas as pl
from jax.experimental.pallas import tpu as pltpu
from jax.experimental.pallas import tpu_sc as plsc

_K = 1024
_BINS = 2 * _K
_NTILES = 32
_CHUNK = 65536
_NPIX = 8 * 512 * 512


def _tc_seg_body(seg_ref, lab_ref, cid_ref, part_ref):
    b = pl.program_id(0)
    q = pl.program_id(1)
    d = seg_ref[0, 1] - seg_ref[0, 0]
    lab = lab_ref[0]
    z = jnp.where(lab == 1, -d, d)
    sp = jnp.maximum(z, 0.0) + jnp.log1p(jnp.exp(-jnp.abs(z)))
    nll = sp
    smooth = 2.0 * sp - z
    e = jnp.exp(z - sp)
    bi = jnp.minimum((e * _K).astype(jnp.int32), _K - 1)
    bi = jnp.maximum(bi, 0)
    cid_ref[0] = bi + _K * lab
    s_nll = jnp.sum(nll)
    s_sm = jnp.sum(smooth)
    first = jnp.logical_and(b == 0, q == 0)

    @pl.when(first)
    def _():
        part_ref[0, 0] = s_nll
        part_ref[0, 1] = s_sm

    @pl.when(jnp.logical_not(first))
    def _():
        part_ref[0, 0] += s_nll
        part_ref[0, 1] += s_sm


def _tc_rec_body(rp_ref, rt_ref, part_ref):
    b = pl.program_id(0)
    s_rec = jnp.sum(jnp.abs(rp_ref[0] - rt_ref[0]))

    @pl.when(b == 0)
    def _():
        part_ref[0, 0] = s_rec

    @pl.when(b != 0)
    def _():
        part_ref[0, 0] += s_rec


def _sc_hist_body(cid_hbm, out_hbm, inbuf, hist, outbuf):
    wid = lax.axis_index("s") * 2 + lax.axis_index("c")
    pltpu.sync_copy(cid_hbm.at[wid], inbuf)
    zeros = jnp.zeros((16,), jnp.float32)
    ones = jnp.ones((16,), jnp.float32)
    lane_base = lax.iota(jnp.int32, 16) * _BINS

    def zbody(i, c):
        for u in range(8):
            hist[pl.ds(i * 128 + u * 16, 16)] = zeros
        return c

    lax.fori_loop(0, (16 * _BINS) // 128, zbody, 0)

    @plsc.parallel_loop(0, _CHUNK // 16, unroll=8)
    def _scatter(i):
        v = inbuf[pl.ds(i * 16, 16)]
        plsc.addupdate_scatter(hist, [v + lane_base], ones)

    def rbody(cch, c):
        acc = hist[pl.ds(cch * 16, 16)]
        for l in range(1, 16):
            acc = acc + hist[pl.ds(l * _BINS + cch * 16, 16)]
        outbuf[pl.ds(cch * 16, 16)] = acc
        return c

    lax.fori_loop(0, _BINS // 16, rbody, 0)
    pltpu.sync_copy(outbuf, out_hbm.at[wid])


@functools.cache
def _sc_hist():
    return pl.kernel(
        _sc_hist_body,
        out_type=jax.ShapeDtypeStruct((_NTILES, _BINS), jnp.float32),
        mesh=plsc.VectorSubcoreMesh(core_axis_name="c", subcore_axis_name="s"),
        scratch_types=[
            pltpu.VMEM((_CHUNK,), jnp.int32),
            pltpu.VMEM((16 * _BINS,), jnp.float32),
            pltpu.VMEM((_BINS,), jnp.float32),
        ],
        compiler_params=pltpu.CompilerParams(needs_layout_passes=False),
    )


def _tc2_body(hist_ref, part_ref, rpart_ref, al_ref, lab_ref, out_ref):
    h = hist_ref[...]
    dot = functools.partial(
        jax.lax.dot_general,
        precision=jax.lax.Precision.HIGHEST,
        preferred_element_type=jnp.float32,
    )
    ti = lax.broadcasted_iota(jnp.int32, (8, _NTILES), 1)
    ri = lax.broadcasted_iota(jnp.int32, (8, _NTILES), 0)
    m = (ti // 4 == ri).astype(jnp.float32)
    him = dot(m, h, (((1,), (0,)), ((), ())))
    h1 = him[:, _K:]
    n = him[:, :_K] + h1
    jj = lax.broadcasted_iota(jnp.int32, (_K, _K), 0)
    kk = lax.broadcasted_iota(jnp.int32, (_K, _K), 1)
    u = (jj >= kk).astype(jnp.float32)
    i_cum = dot(n, u, (((1,), (0,)), ((), ())))
    f1 = dot(h1, u, (((1,), (0,)), ((), ())))
    p1 = f1[:, 0:1]
    ptot = i_cum[:, 0:1]

    def loss_for(f, p):
        denom = jnp.maximum(p + i_cum - f, 1.0)
        jac = 1.0 - (p - f) / denom
        jac = jnp.where(i_cum > 0, jac, 0.0)
        return (jnp.sum(jac, axis=1, keepdims=True) - 0.5 * jac[:, 0:1]) / _K

    l1 = loss_for(f1, p1)
    l0 = loss_for(i_cum - f1, ptot - p1)
    pr1 = (p1 > 0).astype(jnp.float32)
    pr0 = (ptot - p1 > 0).astype(jnp.float32)
    per_img = (pr0 * l0 + pr1 * l1) / jnp.maximum(pr0 + pr1, 1.0)
    lovasz = jnp.sum(per_img) / 8.0
    a = al_ref[...]
    mx = jnp.max(a, axis=1, keepdims=True)
    lse = jnp.log(jnp.sum(jnp.exp(a - mx), axis=1, keepdims=True)) + mx
    logp = a - lse
    colj = lax.broadcasted_iota(jnp.int32, (8, 8), 1)
    pick = jnp.sum(jnp.where(colj == lab_ref[...], logp, 0.0), axis=1)
    align_ce = -jnp.sum(pick) / 8.0
    seg_ce = 0.9 * (part_ref[0, 0] / _NPIX) + 0.05 * (part_ref[0, 1] / _NPIX)
    rec = rpart_ref[0, 0] / (3 * _NPIX)
    out_ref[0, 0] = 3.0 * (seg_ce + lovasz) + align_ce + rec


def kernel(seg_logits, seg_labels, align_logits, align_labels, rec_pred, rec_target):
    cid, partials = pl.pallas_call(
        _tc_seg_body,
        grid=(8, 4),
        in_specs=[
            pl.BlockSpec((1, 2, 128, 512), lambda b, q: (b, 0, q, 0)),
            pl.BlockSpec((1, 128, 512), lambda b, q: (b, q, 0)),
        ],
        out_specs=[
            pl.BlockSpec((1, 128, 512), lambda b, q: (b, q, 0)),
            pl.BlockSpec(memory_space=pltpu.SMEM),
        ],
        out_shape=[
            jax.ShapeDtypeStruct((8, 512, 512), jnp.int32),
            jax.ShapeDtypeStruct((1, 2), jnp.float32),
        ],
    )(seg_logits, seg_labels.astype(jnp.int32))

    hist = _sc_hist()(cid.reshape(_NTILES, _CHUNK))

    rec_partial = pl.pallas_call(
        _tc_rec_body,
        grid=(8,),
        in_specs=[
            pl.BlockSpec((1, 3, 512, 512), lambda b: (b, 0, 0, 0)),
            pl.BlockSpec((1, 3, 512, 512), lambda b: (b, 0, 0, 0)),
        ],
        out_specs=pl.BlockSpec(memory_space=pltpu.SMEM),
        out_shape=jax.ShapeDtypeStruct((1, 1), jnp.float32),
    )(rec_pred, rec_target)

    total = pl.pallas_call(
        _tc2_body,
        in_specs=[
            pl.BlockSpec(memory_space=pltpu.VMEM),
            pl.BlockSpec(memory_space=pltpu.SMEM),
            pl.BlockSpec(memory_space=pltpu.SMEM),
            pl.BlockSpec(memory_space=pltpu.VMEM),
            pl.BlockSpec(memory_space=pltpu.VMEM),
        ],
        out_specs=pl.BlockSpec(memory_space=pltpu.SMEM),
        out_shape=jax.ShapeDtypeStruct((1, 1), jnp.float32),
    )(hist, partials, rec_partial, align_logits,
      align_labels.astype(jnp.int32).reshape(8, 1))
    return total.reshape(())


# speedup vs baseline: 131.0761x; 1.3212x over previous
import functools

import jax
import jax.numpy as jnp
from jax import lax
from jax.experimental import pallas as pl
from jax.experimental.pallas import tpu as pltpu
from jax.experimental.pallas import tpu_sc as plsc

_K = 1024
_BINS = 2 * _K
_NTILES = 32
_CHUNK = 65536
_NPIX = 8 * 512 * 512


def _tc_seg_body(seg_ref, lab_ref, cid_ref, part_ref):
    b = pl.program_id(0)
    d = seg_ref[0, 1] - seg_ref[0, 0]
    lab = lab_ref[0]
    z = jnp.where(lab == 1, -d, d)
    sp = jnp.maximum(z, 0.0) + jnp.log1p(jnp.exp(-jnp.abs(z)))
    nll = sp
    smooth = 2.0 * sp - z
    e = jnp.exp(z - sp)
    bi = jnp.minimum((e * _K).astype(jnp.int32), _K - 1)
    bi = jnp.maximum(bi, 0)
    cid_ref[0] = bi + _K * lab
    s_nll = jnp.sum(nll)
    s_sm = jnp.sum(smooth)

    @pl.when(b == 0)
    def _():
        part_ref[0, 0] = s_nll
        part_ref[0, 1] = s_sm

    @pl.when(b != 0)
    def _():
        part_ref[0, 0] += s_nll
        part_ref[0, 1] += s_sm


def _tc_rec_body(rp_ref, rt_ref, part_ref):
    b = pl.program_id(0)
    s_rec = jnp.sum(jnp.abs(rp_ref[0] - rt_ref[0]))

    @pl.when(b == 0)
    def _():
        part_ref[0, 0] = s_rec

    @pl.when(b != 0)
    def _():
        part_ref[0, 0] += s_rec


def _sc_hist_body(cid_hbm, out_hbm, inbuf, hist, outbuf):
    wid = lax.axis_index("s") * 2 + lax.axis_index("c")
    img = wid // 4
    quarter = wid % 4
    pltpu.sync_copy(cid_hbm.at[img, pl.ds(quarter * 128, 128)], inbuf)
    zeros = jnp.zeros((16,), jnp.float32)
    ones = jnp.ones((16,), jnp.float32)
    lane_base = lax.iota(jnp.int32, 16) * _BINS

    def zbody(i, c):
        for u in range(8):
            hist[pl.ds(i * 128 + u * 16, 16)] = zeros
        return c

    lax.fori_loop(0, (16 * _BINS) // 128, zbody, 0)

    @plsc.parallel_loop(0, _CHUNK // 16, unroll=8)
    def _scatter(i):
        v = inbuf[i // 32, pl.ds((i % 32) * 16, 16)]
        plsc.addupdate_scatter(hist, [v + lane_base], ones)

    def rbody(cch, c):
        acc = hist[pl.ds(cch * 16, 16)]
        for l in range(1, 16):
            acc = acc + hist[pl.ds(l * _BINS + cch * 16, 16)]
        outbuf[pl.ds(cch * 16, 16)] = acc
        return c

    lax.fori_loop(0, _BINS // 16, rbody, 0)
    pltpu.sync_copy(outbuf, out_hbm.at[wid])


@functools.cache
def _sc_hist():
    return pl.kernel(
        _sc_hist_body,
        out_type=jax.ShapeDtypeStruct((_NTILES, _BINS), jnp.float32),
        mesh=plsc.VectorSubcoreMesh(core_axis_name="c", subcore_axis_name="s"),
        scratch_types=[
            pltpu.VMEM((128, 512), jnp.int32),
            pltpu.VMEM((16 * _BINS,), jnp.float32),
            pltpu.VMEM((_BINS,), jnp.float32),
        ],
        compiler_params=pltpu.CompilerParams(needs_layout_passes=False),
    )


def _tc2_body(hist_ref, part_ref, rpart_ref, al_ref, lab_ref, out_ref):
    h = hist_ref[...]
    dot = functools.partial(
        jax.lax.dot_general,
        precision=jax.lax.Precision.HIGHEST,
        preferred_element_type=jnp.float32,
    )
    ti = lax.broadcasted_iota(jnp.int32, (8, _NTILES), 1)
    ri = lax.broadcasted_iota(jnp.int32, (8, _NTILES), 0)
    m = (ti // 4 == ri).astype(jnp.float32)
    him = dot(m, h, (((1,), (0,)), ((), ())))
    h1 = him[:, _K:]
    n = him[:, :_K] + h1
    jj = lax.broadcasted_iota(jnp.int32, (_K, _K), 0)
    kk = lax.broadcasted_iota(jnp.int32, (_K, _K), 1)
    u = (jj >= kk).astype(jnp.float32)
    i_cum = dot(n, u, (((1,), (0,)), ((), ())))
    f1 = dot(h1, u, (((1,), (0,)), ((), ())))
    p1 = f1[:, 0:1]
    ptot = i_cum[:, 0:1]

    def loss_for(f, p):
        denom = jnp.maximum(p + i_cum - f, 1.0)
        jac = 1.0 - (p - f) / denom
        jac = jnp.where(i_cum > 0, jac, 0.0)
        return (jnp.sum(jac, axis=1, keepdims=True) - 0.5 * jac[:, 0:1]) / _K

    l1 = loss_for(f1, p1)
    l0 = loss_for(i_cum - f1, ptot - p1)
    pr1 = (p1 > 0).astype(jnp.float32)
    pr0 = (ptot - p1 > 0).astype(jnp.float32)
    per_img = (pr0 * l0 + pr1 * l1) / jnp.maximum(pr0 + pr1, 1.0)
    lovasz = jnp.sum(per_img) / 8.0
    a = al_ref[...]
    mx = jnp.max(a, axis=1, keepdims=True)
    lse = jnp.log(jnp.sum(jnp.exp(a - mx), axis=1, keepdims=True)) + mx
    logp = a - lse
    colj = lax.broadcasted_iota(jnp.int32, (8, 8), 1)
    pick = jnp.sum(jnp.where(colj == lab_ref[...], logp, 0.0), axis=1)
    align_ce = -jnp.sum(pick) / 8.0
    seg_ce = 0.9 * (part_ref[0, 0] / _NPIX) + 0.05 * (part_ref[0, 1] / _NPIX)
    rec = rpart_ref[0, 0] / (3 * _NPIX)
    out_ref[0, 0] = 3.0 * (seg_ce + lovasz) + align_ce + rec


def kernel(seg_logits, seg_labels, align_logits, align_labels, rec_pred, rec_target):
    cid, partials = pl.pallas_call(
        _tc_seg_body,
        grid=(8,),
        in_specs=[
            pl.BlockSpec((1, 2, 512, 512), lambda b: (b, 0, 0, 0)),
            pl.BlockSpec((1, 512, 512), lambda b: (b, 0, 0)),
        ],
        out_specs=[
            pl.BlockSpec((1, 512, 512), lambda b: (b, 0, 0)),
            pl.BlockSpec(memory_space=pltpu.SMEM),
        ],
        out_shape=[
            jax.ShapeDtypeStruct((8, 512, 512), jnp.int32),
            jax.ShapeDtypeStruct((1, 2), jnp.float32),
        ],
    )(seg_logits, seg_labels.astype(jnp.int32))

    hist = _sc_hist()(cid)

    rec_partial = pl.pallas_call(
        _tc_rec_body,
        grid=(8,),
        in_specs=[
            pl.BlockSpec((1, 3, 512, 512), lambda b: (b, 0, 0, 0)),
            pl.BlockSpec((1, 3, 512, 512), lambda b: (b, 0, 0, 0)),
        ],
        out_specs=pl.BlockSpec(memory_space=pltpu.SMEM),
        out_shape=jax.ShapeDtypeStruct((1, 1), jnp.float32),
    )(rec_pred, rec_target)

    total = pl.pallas_call(
        _tc2_body,
        in_specs=[
            pl.BlockSpec(memory_space=pltpu.VMEM),
            pl.BlockSpec(memory_space=pltpu.SMEM),
            pl.BlockSpec(memory_space=pltpu.SMEM),
            pl.BlockSpec(memory_space=pltpu.VMEM),
            pl.BlockSpec(memory_space=pltpu.VMEM),
        ],
        out_specs=pl.BlockSpec(memory_space=pltpu.SMEM),
        out_shape=jax.ShapeDtypeStruct((1, 1), jnp.float32),
    )(hist, partials, rec_partial, align_logits,
      align_labels.astype(jnp.int32).reshape(8, 1))
    return total.reshape(())


# speedup vs baseline: 133.2592x; 1.0167x over previous
import functools

import jax
import jax.numpy as jnp
from jax import lax
from jax.experimental import pallas as pl
from jax.experimental.pallas import tpu as pltpu
from jax.experimental.pallas import tpu_sc as plsc

_K = 1024
_BINS = 2 * _K
_NTILES = 32
_CHUNK = 65536
_NPIX = 8 * 512 * 512


def _tc_seg_body(seg_ref, lab_ref, cid_ref, part_ref):
    b = pl.program_id(0)
    d = seg_ref[0, 1] - seg_ref[0, 0]
    lab = lab_ref[0]
    z = jnp.where(lab == 1, -d, d)
    sp = jnp.maximum(z, 0.0) + jnp.log1p(jnp.exp(-jnp.abs(z)))
    bi = jnp.minimum(((z + 8.0) * 64.0).astype(jnp.int32), _K - 1)
    bi = jnp.maximum(bi, 0)
    cid_ref[0] = bi + _K * lab
    s_nll = jnp.sum(sp)
    s_z = jnp.sum(z)

    @pl.when(b == 0)
    def _():
        part_ref[0, 0] = s_nll
        part_ref[0, 1] = s_z

    @pl.when(b != 0)
    def _():
        part_ref[0, 0] += s_nll
        part_ref[0, 1] += s_z


def _tc_rec_body(rp_ref, rt_ref, part_ref):
    b = pl.program_id(0)
    s_rec = jnp.sum(jnp.abs(rp_ref[...] - rt_ref[...]))

    @pl.when(b == 0)
    def _():
        part_ref[0, 0] = s_rec

    @pl.when(b != 0)
    def _():
        part_ref[0, 0] += s_rec


def _sc_hist_body(cid_hbm, out_hbm, inbuf, hist, outbuf):
    wid = lax.axis_index("s") * 2 + lax.axis_index("c")
    img = wid // 4
    quarter = wid % 4
    pltpu.sync_copy(cid_hbm.at[img, pl.ds(quarter * 128, 128)], inbuf)
    zeros = jnp.zeros((16,), jnp.float32)
    ones = jnp.ones((16,), jnp.float32)
    lane_base = lax.iota(jnp.int32, 16) * _BINS

    def zbody(i, c):
        for u in range(8):
            hist[pl.ds(i * 128 + u * 16, 16)] = zeros
        return c

    lax.fori_loop(0, (16 * _BINS) // 128, zbody, 0)

    @plsc.parallel_loop(0, _CHUNK // 16, unroll=8)
    def _scatter(i):
        v = inbuf[i // 32, pl.ds((i % 32) * 16, 16)]
        plsc.addupdate_scatter(hist, [v + lane_base], ones)

    def rbody(cch, c):
        acc = hist[pl.ds(cch * 16, 16)]
        for l in range(1, 16):
            acc = acc + hist[pl.ds(l * _BINS + cch * 16, 16)]
        outbuf[pl.ds(cch * 16, 16)] = acc
        return c

    lax.fori_loop(0, _BINS // 16, rbody, 0)
    pltpu.sync_copy(outbuf, out_hbm.at[wid])


@functools.cache
def _sc_hist():
    return pl.kernel(
        _sc_hist_body,
        out_type=jax.ShapeDtypeStruct((_NTILES, _BINS), jnp.float32),
        mesh=plsc.VectorSubcoreMesh(core_axis_name="c", subcore_axis_name="s"),
        scratch_types=[
            pltpu.VMEM((128, 512), jnp.int32),
            pltpu.VMEM((16 * _BINS,), jnp.float32),
            pltpu.VMEM((_BINS,), jnp.float32),
        ],
        compiler_params=pltpu.CompilerParams(needs_layout_passes=False),
    )


def _tc2_body(hist_ref, part_ref, rpart_ref, al_ref, lab_ref, out_ref):
    h = hist_ref[...]
    dot = functools.partial(
        jax.lax.dot_general,
        precision=jax.lax.Precision.HIGHEST,
        preferred_element_type=jnp.float32,
    )
    ti = lax.broadcasted_iota(jnp.int32, (8, _NTILES), 1)
    ri = lax.broadcasted_iota(jnp.int32, (8, _NTILES), 0)
    m = (ti // 4 == ri).astype(jnp.float32)
    him = dot(m, h, (((1,), (0,)), ((), ())))
    h1 = him[:, _K:]
    n = him[:, :_K] + h1
    jj = lax.broadcasted_iota(jnp.int32, (_K, _K), 0)
    kk = lax.broadcasted_iota(jnp.int32, (_K, _K), 1)
    u = (jj >= kk).astype(jnp.float32)
    i_cum = dot(n, u, (((1,), (0,)), ((), ())))
    f1 = dot(h1, u, (((1,), (0,)), ((), ())))
    p1 = f1[:, 0:1]
    ptot = i_cum[:, 0:1]

    mm = lax.broadcasted_iota(jnp.int32, (1, _K), 1)
    zmid = (mm.astype(jnp.float32) + 0.5) / 64.0 - 8.0
    em = 1.0 / (1.0 + jnp.exp(-zmid))
    em_prev = 1.0 / (1.0 + jnp.exp(-(zmid - 1.0 / 64.0)))
    w = em - jnp.where(mm == 0, 0.0, em_prev)

    def loss_for(f, p):
        denom = jnp.maximum(p + i_cum - f, 1.0)
        jac = 1.0 - (p - f) / denom
        jac = jnp.where(i_cum > 0, jac, 0.0)
        return jnp.sum(jac * w, axis=1, keepdims=True)

    l1 = loss_for(f1, p1)
    l0 = loss_for(i_cum - f1, ptot - p1)
    pr1 = (p1 > 0).astype(jnp.float32)
    pr0 = (ptot - p1 > 0).astype(jnp.float32)
    per_img = (pr0 * l0 + pr1 * l1) / jnp.maximum(pr0 + pr1, 1.0)
    lovasz = jnp.sum(per_img) / 8.0
    a = al_ref[...]
    mx = jnp.max(a, axis=1, keepdims=True)
    lse = jnp.log(jnp.sum(jnp.exp(a - mx), axis=1, keepdims=True)) + mx
    logp = a - lse
    colj = lax.broadcasted_iota(jnp.int32, (8, 8), 1)
    pick = jnp.sum(jnp.where(colj == lab_ref[...], logp, 0.0), axis=1)
    align_ce = -jnp.sum(pick) / 8.0
    nll_sum = part_ref[0, 0]
    smooth_sum = 2.0 * nll_sum - part_ref[0, 1]
    seg_ce = 0.9 * (nll_sum / _NPIX) + 0.05 * (smooth_sum / _NPIX)
    rec = rpart_ref[0, 0] / (3 * _NPIX)
    out_ref[0, 0] = 3.0 * (seg_ce + lovasz) + align_ce + rec


def kernel(seg_logits, seg_labels, align_logits, align_labels, rec_pred, rec_target):
    cid, partials = pl.pallas_call(
        _tc_seg_body,
        grid=(8,),
        in_specs=[
            pl.BlockSpec((1, 2, 512, 512), lambda b: (b, 0, 0, 0)),
            pl.BlockSpec((1, 512, 512), lambda b: (b, 0, 0)),
        ],
        out_specs=[
            pl.BlockSpec((1, 512, 512), lambda b: (b, 0, 0)),
            pl.BlockSpec(memory_space=pltpu.SMEM),
        ],
        out_shape=[
            jax.ShapeDtypeStruct((8, 512, 512), jnp.int32),
            jax.ShapeDtypeStruct((1, 2), jnp.float32),
        ],
    )(seg_logits, seg_labels.astype(jnp.int32))

    hist = _sc_hist()(cid)

    rec_partial = pl.pallas_call(
        _tc_rec_body,
        grid=(4,),
        in_specs=[
            pl.BlockSpec((2, 3, 512, 512), lambda b: (b, 0, 0, 0)),
            pl.BlockSpec((2, 3, 512, 512), lambda b: (b, 0, 0, 0)),
        ],
        out_specs=pl.BlockSpec(memory_space=pltpu.SMEM),
        out_shape=jax.ShapeDtypeStruct((1, 1), jnp.float32),
    )(rec_pred, rec_target)

    total = pl.pallas_call(
        _tc2_body,
        in_specs=[
            pl.BlockSpec(memory_space=pltpu.VMEM),
            pl.BlockSpec(memory_space=pltpu.SMEM),
            pl.BlockSpec(memory_space=pltpu.SMEM),
            pl.BlockSpec(memory_space=pltpu.VMEM),
            pl.BlockSpec(memory_space=pltpu.VMEM),
        ],
        out_specs=pl.BlockSpec(memory_space=pltpu.SMEM),
        out_shape=jax.ShapeDtypeStruct((1, 1), jnp.float32),
    )(hist, partials, rec_partial, align_logits,
      align_labels.astype(jnp.int32).reshape(8, 1))
    return total.reshape(())


# speedup vs baseline: 138.0655x; 1.0361x over previous
import functools

import jax
import jax.numpy as jnp
from jax import lax
from jax.experimental import pallas as pl
from jax.experimental.pallas import tpu as pltpu
from jax.experimental.pallas import tpu_sc as plsc

_K = 1024
_BINS = 2 * _K
_NTILES = 32
_CHUNK = 65536
_NPIX = 8 * 512 * 512


def _tc_seg_body(seg_ref, lab_ref, cid_ref, part_ref):
    b = pl.program_id(0)
    d = seg_ref[:, 1] - seg_ref[:, 0]
    lab = lab_ref[...]
    z = jnp.where(lab == 1, -d, d)
    sp = jnp.maximum(z, 0.0) + jnp.log1p(jnp.exp(-jnp.abs(z)))
    bi = jnp.minimum(((z + 8.0) * 64.0).astype(jnp.int32), _K - 1)
    bi = jnp.maximum(bi, 0)
    cid_ref[...] = bi + _K * lab
    s_nll = jnp.sum(sp)
    s_z = jnp.sum(z)

    @pl.when(b == 0)
    def _():
        part_ref[0, 0] = s_nll
        part_ref[0, 1] = s_z

    @pl.when(b != 0)
    def _():
        part_ref[0, 0] += s_nll
        part_ref[0, 1] += s_z


def _tc_rec_body(rp_ref, rt_ref, part_ref):
    b = pl.program_id(0)
    s_rec = jnp.sum(jnp.abs(rp_ref[...] - rt_ref[...]))

    @pl.when(b == 0)
    def _():
        part_ref[0, 0] = s_rec

    @pl.when(b != 0)
    def _():
        part_ref[0, 0] += s_rec


def _sc_hist_body(cid_hbm, out_hbm, inbuf, hist, outbuf):
    wid = lax.axis_index("s") * 2 + lax.axis_index("c")
    img = wid // 4
    quarter = wid % 4
    pltpu.sync_copy(cid_hbm.at[img, pl.ds(quarter * 128, 128)], inbuf)
    zeros = jnp.zeros((16,), jnp.float32)
    ones = jnp.ones((16,), jnp.float32)
    lane_base = lax.iota(jnp.int32, 16) * _BINS

    def zbody(i, c):
        for u in range(8):
            hist[pl.ds(i * 128 + u * 16, 16)] = zeros
        return c

    lax.fori_loop(0, (16 * _BINS) // 128, zbody, 0)

    @plsc.parallel_loop(0, _CHUNK // 16, unroll=8)
    def _scatter(i):
        v = inbuf[i // 32, pl.ds((i % 32) * 16, 16)]
        plsc.addupdate_scatter(hist, [v + lane_base], ones)

    def rbody(cch, c):
        acc = hist[pl.ds(cch * 16, 16)]
        for l in range(1, 16):
            acc = acc + hist[pl.ds(l * _BINS + cch * 16, 16)]
        outbuf[pl.ds(cch * 16, 16)] = acc
        return c

    lax.fori_loop(0, _BINS // 16, rbody, 0)
    pltpu.sync_copy(outbuf, out_hbm.at[wid])


@functools.cache
def _sc_hist():
    return pl.kernel(
        _sc_hist_body,
        out_type=jax.ShapeDtypeStruct((_NTILES, _BINS), jnp.float32),
        mesh=plsc.VectorSubcoreMesh(core_axis_name="c", subcore_axis_name="s"),
        scratch_types=[
            pltpu.VMEM((128, 512), jnp.int32),
            pltpu.VMEM((16 * _BINS,), jnp.float32),
            pltpu.VMEM((_BINS,), jnp.float32),
        ],
        compiler_params=pltpu.CompilerParams(needs_layout_passes=False),
    )


def _tc2_body(hist_ref, part_ref, rpart_ref, al_ref, lab_ref, out_ref):
    h = hist_ref[...]
    dot = functools.partial(
        jax.lax.dot_general,
        precision=jax.lax.Precision.HIGHEST,
        preferred_element_type=jnp.float32,
    )
    ti = lax.broadcasted_iota(jnp.int32, (8, _NTILES), 1)
    ri = lax.broadcasted_iota(jnp.int32, (8, _NTILES), 0)
    m = (ti // 4 == ri).astype(jnp.float32)
    him = dot(m, h, (((1,), (0,)), ((), ())))
    h1 = him[:, _K:]
    n = him[:, :_K] + h1

    def suffix_sum(x):
        sh = 1
        while sh < _K:
            x = x + jnp.concatenate(
                [x[:, sh:], jnp.zeros((8, sh), jnp.float32)], axis=1)
            sh *= 2
        return x

    i_cum = suffix_sum(n)
    f1 = suffix_sum(h1)
    p1 = f1[:, 0:1]
    ptot = i_cum[:, 0:1]

    mm = lax.broadcasted_iota(jnp.int32, (1, _K), 1)
    zmid = (mm.astype(jnp.float32) + 0.5) / 64.0 - 8.0
    em = 1.0 / (1.0 + jnp.exp(-zmid))
    em_prev = 1.0 / (1.0 + jnp.exp(-(zmid - 1.0 / 64.0)))
    w = em - jnp.where(mm == 0, 0.0, em_prev)

    def loss_for(f, p):
        denom = jnp.maximum(p + i_cum - f, 1.0)
        jac = 1.0 - (p - f) / denom
        jac = jnp.where(i_cum > 0, jac, 0.0)
        return jnp.sum(jac * w, axis=1, keepdims=True)

    l1 = loss_for(f1, p1)
    l0 = loss_for(i_cum - f1, ptot - p1)
    pr1 = (p1 > 0).astype(jnp.float32)
    pr0 = (ptot - p1 > 0).astype(jnp.float32)
    per_img = (pr0 * l0 + pr1 * l1) / jnp.maximum(pr0 + pr1, 1.0)
    lovasz = jnp.sum(per_img) / 8.0
    a = al_ref[...]
    mx = jnp.max(a, axis=1, keepdims=True)
    lse = jnp.log(jnp.sum(jnp.exp(a - mx), axis=1, keepdims=True)) + mx
    logp = a - lse
    colj = lax.broadcasted_iota(jnp.int32, (8, 8), 1)
    pick = jnp.sum(jnp.where(colj == lab_ref[...], logp, 0.0), axis=1)
    align_ce = -jnp.sum(pick) / 8.0
    nll_sum = part_ref[0, 0]
    smooth_sum = 2.0 * nll_sum - part_ref[0, 1]
    seg_ce = 0.9 * (nll_sum / _NPIX) + 0.05 * (smooth_sum / _NPIX)
    rec = rpart_ref[0, 0] / (3 * _NPIX)
    out_ref[0, 0] = 3.0 * (seg_ce + lovasz) + align_ce + rec


def kernel(seg_logits, seg_labels, align_logits, align_labels, rec_pred, rec_target):
    cid, partials = pl.pallas_call(
        _tc_seg_body,
        grid=(4,),
        in_specs=[
            pl.BlockSpec((2, 2, 512, 512), lambda b: (b, 0, 0, 0)),
            pl.BlockSpec((2, 512, 512), lambda b: (b, 0, 0)),
        ],
        out_specs=[
            pl.BlockSpec((2, 512, 512), lambda b: (b, 0, 0)),
            pl.BlockSpec(memory_space=pltpu.SMEM),
        ],
        out_shape=[
            jax.ShapeDtypeStruct((8, 512, 512), jnp.int32),
            jax.ShapeDtypeStruct((1, 2), jnp.float32),
        ],
    )(seg_logits, seg_labels.astype(jnp.int32))

    hist = _sc_hist()(cid)

    rec_partial = pl.pallas_call(
        _tc_rec_body,
        grid=(4,),
        in_specs=[
            pl.BlockSpec((2, 3, 512, 512), lambda b: (b, 0, 0, 0)),
            pl.BlockSpec((2, 3, 512, 512), lambda b: (b, 0, 0, 0)),
        ],
        out_specs=pl.BlockSpec(memory_space=pltpu.SMEM),
        out_shape=jax.ShapeDtypeStruct((1, 1), jnp.float32),
    )(rec_pred, rec_target)

    total = pl.pallas_call(
        _tc2_body,
        in_specs=[
            pl.BlockSpec(memory_space=pltpu.VMEM),
            pl.BlockSpec(memory_space=pltpu.SMEM),
            pl.BlockSpec(memory_space=pltpu.SMEM),
            pl.BlockSpec(memory_space=pltpu.VMEM),
            pl.BlockSpec(memory_space=pltpu.VMEM),
        ],
        out_specs=pl.BlockSpec(memory_space=pltpu.SMEM),
        out_shape=jax.ShapeDtypeStruct((1, 1), jnp.float32),
    )(hist, partials, rec_partial, align_logits,
      align_labels.astype(jnp.int32).reshape(8, 1))
    return total.reshape(())
